# baseline (speedup 1.0000x reference)
import jax
import jax.numpy as jnp
from jax.experimental import pallas as pl
from jax.experimental.pallas import tpu as pltpu
from jax.experimental.pallas import tpu_sc as plsc

_TOPK = 5
_TILE = 12288
_CARRY_W = 8
_IDX_BIG = jnp.iinfo(jnp.int32).max
_LANES = 128


def _pack_carry(new_vals, new_idx, n):
    pad = _CARRY_W - _TOPK
    cv_new = jnp.concatenate(
        new_vals + [jnp.full((n, pad), -jnp.inf, jnp.float32)], axis=1
    )
    ci_new = jnp.concatenate(new_idx + [jnp.zeros((n, pad), jnp.int32)], axis=1)
    return cv_new, ci_new


def _carry_round(cv, ci, mv_t, pick_t, base):
    mv_c = jnp.max(cv, axis=1, keepdims=True)
    i_c = jnp.min(jnp.where(cv == mv_c, ci, _IDX_BIG), axis=1, keepdims=True)
    use_c = mv_c >= mv_t
    m = jnp.where(use_c, mv_c, mv_t)
    pick = jnp.where(use_c, i_c, pick_t + base)
    return use_c, m, pick


def _merge_exact(sim, cv, ci, base):
    col = jax.lax.broadcasted_iota(jnp.int32, sim.shape, 1)
    new_vals, new_idx = [], []
    for _ in range(_TOPK):
        mv_t = jnp.max(sim, axis=1, keepdims=True)
        am_t = jnp.argmax(sim, axis=1).astype(jnp.int32)[:, None]
        use_c, m, pick = _carry_round(cv, ci, mv_t, am_t, base)
        new_vals.append(m)
        new_idx.append(pick)
        kill_t = jnp.where(use_c, jnp.int32(-1), am_t)
        sim = jnp.where(col == kill_t, -jnp.inf, sim)
        cv = jnp.where(ci == pick, -jnp.inf, cv)
    return _pack_carry(new_vals, new_idx, cv.shape[0])


def _merge_fold(sim, cv, ci, base):
    n, t = sim.shape
    lane = jax.lax.broadcasted_iota(jnp.int32, (n, _LANES), 1)
    y1 = sim[:, 0:_LANES]
    x1 = lane
    y2 = jnp.full((n, _LANES), -jnp.inf, jnp.float32)
    x2 = jnp.zeros((n, _LANES), jnp.int32)
    y3 = y2
    x3 = x2
    for g in range(1, t // _LANES):
        s = sim[:, g * _LANES:(g + 1) * _LANES]
        c = lane + g * _LANES
        gt1 = s > y1
        l1v = jnp.where(gt1, y1, s)
        l1x = jnp.where(gt1, x1, c)
        gt2 = (l1v > y2) | ((l1v == y2) & (l1x < x2))
        l2v = jnp.where(gt2, y2, l1v)
        l2x = jnp.where(gt2, x2, l1x)
        gt3 = (l2v > y3) | ((l2v == y3) & (l2x < x3))
        y1 = jnp.where(gt1, s, y1)
        x1 = jnp.where(gt1, c, x1)
        y2 = jnp.where(gt2, l1v, y2)
        x2 = jnp.where(gt2, l1x, x2)
        y3 = jnp.where(gt3, l2v, y3)
        x3 = jnp.where(gt3, l2x, x3)
    trig_m = jnp.zeros((n, _LANES), jnp.bool_)
    new_vals, new_idx = [], []
    for r in range(_TOPK):
        mv_t = jnp.max(y1, axis=1, keepdims=True)
        pick_t = jnp.min(
            jnp.where(y1 == mv_t, x1, _IDX_BIG), axis=1, keepdims=True
        )
        use_c, m, pick = _carry_round(cv, ci, mv_t, pick_t, base)
        new_vals.append(m)
        new_idx.append(pick)
        if r < _TOPK - 1:
            kill_t = jnp.where(use_c, jnp.int32(-1), pick_t)
            sel = x1 == kill_t
            trig_m = trig_m | (sel & (y2 == -jnp.inf))
            y1 = jnp.where(sel, y2, y1)
            x1 = jnp.where(sel, x2, x1)
            y2 = jnp.where(sel, y3, y2)
            x2 = jnp.where(sel, x3, x2)
            y3 = jnp.where(sel, -jnp.inf, y3)
        cv = jnp.where(ci == pick, -jnp.inf, cv)
    cv_new, ci_new = _pack_carry(new_vals, new_idx, n)
    return cv_new, ci_new, jnp.any(trig_m)


def _scan_body(out_ref, bank_ref, bn_ref, idx_ref, cv_ref, ci_ref):
    i = pl.program_id(0)
    n_steps = pl.num_programs(0)

    @pl.when(i == 0)
    def _():
        cv_ref[...] = jnp.full(cv_ref.shape, -jnp.inf, jnp.float32)
        ci_ref[...] = jnp.zeros(ci_ref.shape, jnp.int32)

    b = bank_ref[...]
    sq = jnp.sum(b * b, axis=1, keepdims=True)
    bn = b * jax.lax.rsqrt(jnp.maximum(sq, 1e-24))
    bn_ref[...] = bn

    q = out_ref[...]
    qnorm = jnp.sqrt(jnp.sum(q * q, axis=1, keepdims=True))
    qn = q / jnp.maximum(qnorm, 1e-12)

    sim = jax.lax.dot_general(
        qn, bn, (((1,), (1,)), ((), ())), preferred_element_type=jnp.float32
    )

    base = i * bank_ref.shape[0]
    cv0 = cv_ref[...]
    ci0 = ci_ref[...]
    cv_new, ci_new, trig = _merge_fold(sim, cv0, ci0, base)
    cv_ref[...] = cv_new
    ci_ref[...] = ci_new

    @pl.when(trig)
    def _():
        cv_e, ci_e = _merge_exact(sim, cv0, ci0, base)
        cv_ref[...] = cv_e
        ci_ref[...] = ci_e

    @pl.when(i == n_steps - 1)
    def _():
        idx_ref[...] = ci_ref[...]


def _normalize_and_topk(output, bank):
    n_queries, dim = output.shape
    n_rows = bank.shape[0]
    n_tiles = n_rows // _TILE
    bank_normed, idx = pl.pallas_call(
        _scan_body,
        grid=(n_tiles,),
        in_specs=[
            pl.BlockSpec((n_queries, dim), lambda i: (0, 0)),
            pl.BlockSpec((_TILE, dim), lambda i: (i, 0)),
        ],
        out_specs=[
            pl.BlockSpec((_TILE, dim), lambda i: (i, 0)),
            pl.BlockSpec((n_queries, _CARRY_W), lambda i: (0, 0)),
        ],
        out_shape=[
            jax.ShapeDtypeStruct((n_rows, dim), jnp.float32),
            jax.ShapeDtypeStruct((n_queries, _CARRY_W), jnp.int32),
        ],
        scratch_shapes=[
            pltpu.VMEM((n_queries, _CARRY_W), jnp.float32),
            pltpu.VMEM((n_queries, _CARRY_W), jnp.int32),
        ],
        compiler_params=pltpu.CompilerParams(
            dimension_semantics=("arbitrary",),
        ),
    )(output, bank)
    return bank_normed, idx


def _copy_row(dst_ref, dst_row, src_ref, src_row, dim):
    for j in range(dim // 16):
        sl = pl.ds(j * 16, 16)
        dst_ref[dst_row, sl] = src_ref[src_row, sl]


def _sc_gather_assemble(bank, output, idx_padded):
    n_q, dim = output.shape
    qps = 8
    mesh = plsc.VectorSubcoreMesh(core_axis_name="core", subcore_axis_name="subcore")

    @pl.kernel(
        out_type=[
            jax.ShapeDtypeStruct((n_q * _TOPK, dim), bank.dtype),
            jax.ShapeDtypeStruct((n_q * (_TOPK + 1), dim), bank.dtype),
        ],
        mesh=mesh,
        scratch_types=[
            pltpu.VMEM((qps, _CARRY_W), jnp.int32),
            pltpu.VMEM((qps * _CARRY_W, 256), jnp.float32),
            pltpu.VMEM((qps * _TOPK, 256), jnp.float32),
            pltpu.VMEM((qps * (_TOPK + 1), 256), jnp.float32),
            pltpu.VMEM((qps, 256), jnp.float32),
        ],
    )
    def gather_kernel(bank_hbm, out_hbm, i_hbm, nn_hbm, res_hbm,
                      idx_vmem, g64, nnb, resb, ob):
        sid = jax.lax.axis_index("core") * 16 + jax.lax.axis_index("subcore")

        def stage_and_gather(s):
            pltpu.sync_copy(i_hbm.at[pl.ds(qps * s, qps)], idx_vmem)
            for k in range(qps):
                pltpu.sync_copy(
                    bank_hbm.at[idx_vmem.at[k]],
                    g64.at[pl.ds(k * _CARRY_W, _CARRY_W)],
                )

        @pl.when(sid < 8)
        def _():
            s = sid
            stage_and_gather(s)
            for k in range(qps):
                for r in range(_TOPK):
                    _copy_row(nnb, k * _TOPK + r, g64, k * _CARRY_W + r, dim)
            pltpu.sync_copy(
                nnb, nn_hbm.at[pl.ds(qps * _TOPK * s, qps * _TOPK)]
            )

        @pl.when((sid >= 8) & (sid < 16))
        def _():
            s = sid - 8
            stage_and_gather(s)
            pltpu.sync_copy(out_hbm.at[pl.ds(qps * s, qps)], ob)
            for k in range(qps):
                _copy_row(resb, k * 6, ob, k, dim)
                for r in range(_TOPK):
                    _copy_row(resb, k * 6 + 1 + r, g64, k * _CARRY_W + r, dim)
            pltpu.sync_copy(resb, res_hbm.at[pl.ds(qps * 6 * s, qps * 6)])

    return gather_kernel(bank, output, idx_padded)


def kernel(output, bank):
    bank_normed, idx_padded = _normalize_and_topk(output, bank)
    nearest_neighbours, res = _sc_gather_assemble(bank, output, idx_padded)
    return (nearest_neighbours, res, output, bank_normed)


# speedup vs baseline: 1.0013x; 1.0013x over previous
import jax
import jax.numpy as jnp
from jax.experimental import pallas as pl
from jax.experimental.pallas import tpu as pltpu
from jax.experimental.pallas import tpu_sc as plsc

_TOPK = 5
_TILE = 12288
_CARRY_W = 8
_IDX_BIG = jnp.iinfo(jnp.int32).max
_LANES = 128


def _pack_carry(new_vals, new_idx, n):
    pad = _CARRY_W - _TOPK
    cv_new = jnp.concatenate(
        new_vals + [jnp.full((n, pad), -jnp.inf, jnp.float32)], axis=1
    )
    ci_new = jnp.concatenate(new_idx + [jnp.zeros((n, pad), jnp.int32)], axis=1)
    return cv_new, ci_new


def _carry_round(cv, ci, mv_t, pick_t, base):
    mv_c = jnp.max(cv, axis=1, keepdims=True)
    i_c = jnp.min(jnp.where(cv == mv_c, ci, _IDX_BIG), axis=1, keepdims=True)
    use_c = mv_c >= mv_t
    m = jnp.where(use_c, mv_c, mv_t)
    pick = jnp.where(use_c, i_c, pick_t + base)
    return use_c, m, pick


def _merge_exact(sim, cv, ci, base):
    col = jax.lax.broadcasted_iota(jnp.int32, sim.shape, 1)
    new_vals, new_idx = [], []
    for _ in range(_TOPK):
        mv_t = jnp.max(sim, axis=1, keepdims=True)
        am_t = jnp.argmax(sim, axis=1).astype(jnp.int32)[:, None]
        use_c, m, pick = _carry_round(cv, ci, mv_t, am_t, base)
        new_vals.append(m)
        new_idx.append(pick)
        kill_t = jnp.where(use_c, jnp.int32(-1), am_t)
        sim = jnp.where(col == kill_t, -jnp.inf, sim)
        cv = jnp.where(ci == pick, -jnp.inf, cv)
    return _pack_carry(new_vals, new_idx, cv.shape[0])


def _merge_fold(sim, cv, ci, base):
    n, t = sim.shape
    lane = jax.lax.broadcasted_iota(jnp.int32, (n, _LANES), 1)
    y1 = sim[:, 0:_LANES]
    x1 = lane
    y2 = jnp.full((n, _LANES), -jnp.inf, jnp.float32)
    x2 = jnp.zeros((n, _LANES), jnp.int32)
    y3 = y2
    x3 = x2
    for g in range(1, t // _LANES):
        s = sim[:, g * _LANES:(g + 1) * _LANES]
        c = lane + g * _LANES
        gt1 = s > y1
        l1v = jnp.where(gt1, y1, s)
        l1x = jnp.where(gt1, x1, c)
        gt2 = (l1v > y2) | ((l1v == y2) & (l1x < x2))
        l2v = jnp.where(gt2, y2, l1v)
        l2x = jnp.where(gt2, x2, l1x)
        gt3 = (l2v > y3) | ((l2v == y3) & (l2x < x3))
        y1 = jnp.where(gt1, s, y1)
        x1 = jnp.where(gt1, c, x1)
        y2 = jnp.where(gt2, l1v, y2)
        x2 = jnp.where(gt2, l1x, x2)
        y3 = jnp.where(gt3, l2v, y3)
        x3 = jnp.where(gt3, l2x, x3)
    trig_m = jnp.zeros((n, _LANES), jnp.bool_)
    new_vals, new_idx = [], []
    for r in range(_TOPK):
        mv_t = jnp.max(y1, axis=1, keepdims=True)
        pick_t = jnp.min(
            jnp.where(y1 == mv_t, x1, _IDX_BIG), axis=1, keepdims=True
        )
        use_c, m, pick = _carry_round(cv, ci, mv_t, pick_t, base)
        new_vals.append(m)
        new_idx.append(pick)
        if r < _TOPK - 1:
            kill_t = jnp.where(use_c, jnp.int32(-1), pick_t)
            sel = x1 == kill_t
            trig_m = trig_m | (sel & (y2 == -jnp.inf))
            y1 = jnp.where(sel, y2, y1)
            x1 = jnp.where(sel, x2, x1)
            y2 = jnp.where(sel, y3, y2)
            x2 = jnp.where(sel, x3, x2)
            y3 = jnp.where(sel, -jnp.inf, y3)
        cv = jnp.where(ci == pick, -jnp.inf, cv)
    cv_new, ci_new = _pack_carry(new_vals, new_idx, n)
    return cv_new, ci_new, jnp.any(trig_m)


def _scan_body(out_ref, bank_ref, bn_ref, idx_ref, cv_ref, ci_ref):
    i = pl.program_id(0)
    n_steps = pl.num_programs(0)

    @pl.when(i == 0)
    def _():
        cv_ref[...] = jnp.full(cv_ref.shape, -jnp.inf, jnp.float32)
        ci_ref[...] = jnp.zeros(ci_ref.shape, jnp.int32)

    b = bank_ref[...]
    sq = jnp.sum(b * b, axis=1, keepdims=True)
    bn = b * jax.lax.rsqrt(jnp.maximum(sq, 1e-24))
    bn_ref[...] = bn

    q = out_ref[...]
    qnorm = jnp.sqrt(jnp.sum(q * q, axis=1, keepdims=True))
    qn = q / jnp.maximum(qnorm, 1e-12)

    sim = jax.lax.dot_general(
        qn, bn, (((1,), (1,)), ((), ())), preferred_element_type=jnp.float32
    )

    base = i * bank_ref.shape[0]
    cv0 = cv_ref[...]
    ci0 = ci_ref[...]
    cv_new, ci_new, trig = _merge_fold(sim, cv0, ci0, base)
    cv_ref[...] = cv_new
    ci_ref[...] = ci_new

    @pl.when(trig)
    def _():
        cv_e, ci_e = _merge_exact(sim, cv0, ci0, base)
        cv_ref[...] = cv_e
        ci_ref[...] = ci_e

    @pl.when(i == n_steps - 1)
    def _():
        idx_ref[...] = ci_ref[...]


def _normalize_and_topk(output, bank):
    n_queries, dim = output.shape
    n_rows = bank.shape[0]
    n_tiles = n_rows // _TILE
    bank_normed, idx = pl.pallas_call(
        _scan_body,
        grid=(n_tiles,),
        in_specs=[
            pl.BlockSpec((n_queries, dim), lambda i: (0, 0)),
            pl.BlockSpec((_TILE, dim), lambda i: (i, 0)),
        ],
        out_specs=[
            pl.BlockSpec((_TILE, dim), lambda i: (i, 0)),
            pl.BlockSpec((n_queries, _CARRY_W), lambda i: (0, 0)),
        ],
        out_shape=[
            jax.ShapeDtypeStruct((n_rows, dim), jnp.float32),
            jax.ShapeDtypeStruct((n_queries, _CARRY_W), jnp.int32),
        ],
        scratch_shapes=[
            pltpu.VMEM((n_queries, _CARRY_W), jnp.float32),
            pltpu.VMEM((n_queries, _CARRY_W), jnp.int32),
        ],
        compiler_params=pltpu.CompilerParams(
            dimension_semantics=("arbitrary",),
        ),
    )(output, bank)
    return bank_normed, idx


def _copy_row(dst_ref, dst_row, src_ref, src_row, dim):
    for j in range(dim // 16):
        sl = pl.ds(j * 16, 16)
        dst_ref[dst_row, sl] = src_ref[src_row, sl]


def _sc_gather_assemble(bank, output, idx_padded):
    n_q, dim = output.shape
    qps = 8
    mesh = plsc.VectorSubcoreMesh(core_axis_name="core", subcore_axis_name="subcore")

    @pl.kernel(
        out_type=[
            jax.ShapeDtypeStruct((n_q * _TOPK, dim), bank.dtype),
            jax.ShapeDtypeStruct((n_q * (_TOPK + 1), dim), bank.dtype),
        ],
        mesh=mesh,
        scratch_types=[
            pltpu.VMEM((qps, _CARRY_W), jnp.int32),
            pltpu.VMEM((qps * _CARRY_W, 256), jnp.float32),
            pltpu.VMEM((qps * _TOPK, 256), jnp.float32),
            pltpu.VMEM((qps * (_TOPK + 1), 256), jnp.float32),
            pltpu.VMEM((qps, 256), jnp.float32),
            pltpu.SemaphoreType.DMA,
        ],
    )
    def gather_kernel(bank_hbm, out_hbm, i_hbm, nn_hbm, res_hbm,
                      idx_vmem, g64, nnb, resb, ob, sem):
        sid = jax.lax.axis_index("core") * 16 + jax.lax.axis_index("subcore")

        def stage_and_gather(s):
            pltpu.sync_copy(i_hbm.at[pl.ds(qps * s, qps)], idx_vmem)
            copies = [
                pltpu.make_async_copy(
                    bank_hbm.at[idx_vmem.at[k]],
                    g64.at[pl.ds(k * _CARRY_W, _CARRY_W)],
                    sem,
                )
                for k in range(qps)
            ]
            for cp in copies:
                cp.start()
            for cp in copies:
                cp.wait()

        @pl.when(sid < 8)
        def _():
            s = sid
            stage_and_gather(s)
            for k in range(qps):
                for r in range(_TOPK):
                    _copy_row(nnb, k * _TOPK + r, g64, k * _CARRY_W + r, dim)
            pltpu.sync_copy(
                nnb, nn_hbm.at[pl.ds(qps * _TOPK * s, qps * _TOPK)]
            )

        @pl.when((sid >= 8) & (sid < 16))
        def _():
            s = sid - 8
            stage_and_gather(s)
            pltpu.sync_copy(out_hbm.at[pl.ds(qps * s, qps)], ob)
            for k in range(qps):
                _copy_row(resb, k * 6, ob, k, dim)
                for r in range(_TOPK):
                    _copy_row(resb, k * 6 + 1 + r, g64, k * _CARRY_W + r, dim)
            pltpu.sync_copy(resb, res_hbm.at[pl.ds(qps * 6 * s, qps * 6)])

    return gather_kernel(bank, output, idx_padded)


def kernel(output, bank):
    bank_normed, idx_padded = _normalize_and_topk(output, bank)
    nearest_neighbours, res = _sc_gather_assemble(bank, output, idx_padded)
    return (nearest_neighbours, res, output, bank_normed)


# speedup vs baseline: 1.0545x; 1.0531x over previous
import jax
import jax.numpy as jnp
from jax.experimental import pallas as pl
from jax.experimental.pallas import tpu as pltpu
from jax.experimental.pallas import tpu_sc as plsc

_TOPK = 5
_TILE = 12288
_CARRY_W = 8
_IDX_BIG = jnp.iinfo(jnp.int32).max
_LANES = 128


def _pack_carry(new_vals, new_idx, n):
    pad = _CARRY_W - _TOPK
    cv_new = jnp.concatenate(
        new_vals + [jnp.full((n, pad), -jnp.inf, jnp.float32)], axis=1
    )
    ci_new = jnp.concatenate(new_idx + [jnp.zeros((n, pad), jnp.int32)], axis=1)
    return cv_new, ci_new


def _carry_round(cv, ci, mv_t, pick_t, base):
    mv_c = jnp.max(cv, axis=1, keepdims=True)
    i_c = jnp.min(jnp.where(cv == mv_c, ci, _IDX_BIG), axis=1, keepdims=True)
    use_c = mv_c >= mv_t
    m = jnp.where(use_c, mv_c, mv_t)
    pick = jnp.where(use_c, i_c, pick_t + base)
    return use_c, m, pick


def _merge_exact(sim, cv, ci, base):
    col = jax.lax.broadcasted_iota(jnp.int32, sim.shape, 1)
    new_vals, new_idx = [], []
    for _ in range(_TOPK):
        mv_t = jnp.max(sim, axis=1, keepdims=True)
        am_t = jnp.argmax(sim, axis=1).astype(jnp.int32)[:, None]
        use_c, m, pick = _carry_round(cv, ci, mv_t, am_t, base)
        new_vals.append(m)
        new_idx.append(pick)
        kill_t = jnp.where(use_c, jnp.int32(-1), am_t)
        sim = jnp.where(col == kill_t, -jnp.inf, sim)
        cv = jnp.where(ci == pick, -jnp.inf, cv)
    return _pack_carry(new_vals, new_idx, cv.shape[0])


def _merge_fold(sim, cv, ci, base):
    n, t = sim.shape
    lane = jax.lax.broadcasted_iota(jnp.int32, (n, _LANES), 1)
    y1 = sim[:, 0:_LANES]
    x1 = lane
    y2 = jnp.full((n, _LANES), -jnp.inf, jnp.float32)
    x2 = jnp.zeros((n, _LANES), jnp.int32)
    y3 = y2
    x3 = x2
    for g in range(1, t // _LANES):
        s = sim[:, g * _LANES:(g + 1) * _LANES]
        c = lane + g * _LANES
        gt1 = s > y1
        l1v = jnp.where(gt1, y1, s)
        l1x = jnp.where(gt1, x1, c)
        gt2 = (l1v > y2) | ((l1v == y2) & (l1x < x2))
        l2v = jnp.where(gt2, y2, l1v)
        l2x = jnp.where(gt2, x2, l1x)
        gt3 = (l2v > y3) | ((l2v == y3) & (l2x < x3))
        y1 = jnp.where(gt1, s, y1)
        x1 = jnp.where(gt1, c, x1)
        y2 = jnp.where(gt2, l1v, y2)
        x2 = jnp.where(gt2, l1x, x2)
        y3 = jnp.where(gt3, l2v, y3)
        x3 = jnp.where(gt3, l2x, x3)
    trig_m = jnp.zeros((n, _LANES), jnp.bool_)
    new_vals, new_idx = [], []
    for r in range(_TOPK):
        mv_t = jnp.max(y1, axis=1, keepdims=True)
        pick_t = jnp.min(
            jnp.where(y1 == mv_t, x1, _IDX_BIG), axis=1, keepdims=True
        )
        use_c, m, pick = _carry_round(cv, ci, mv_t, pick_t, base)
        new_vals.append(m)
        new_idx.append(pick)
        if r < _TOPK - 1:
            kill_t = jnp.where(use_c, jnp.int32(-1), pick_t)
            sel = x1 == kill_t
            trig_m = trig_m | (sel & (y2 == -jnp.inf))
            y1 = jnp.where(sel, y2, y1)
            x1 = jnp.where(sel, x2, x1)
            y2 = jnp.where(sel, y3, y2)
            x2 = jnp.where(sel, x3, x2)
            y3 = jnp.where(sel, -jnp.inf, y3)
        cv = jnp.where(ci == pick, -jnp.inf, cv)
    cv_new, ci_new = _pack_carry(new_vals, new_idx, n)
    return cv_new, ci_new, jnp.any(trig_m)


def _scan_body(out_ref, bank_ref, bn_ref, idx_ref, cv_ref, ci_ref):
    i = pl.program_id(0)
    n_steps = pl.num_programs(0)

    @pl.when(i == 0)
    def _():
        cv_ref[...] = jnp.full(cv_ref.shape, -jnp.inf, jnp.float32)
        ci_ref[...] = jnp.zeros(ci_ref.shape, jnp.int32)

    b = bank_ref[...]
    sq = jnp.sum(b * b, axis=1, keepdims=True)
    bn = b * jax.lax.rsqrt(jnp.maximum(sq, 1e-24))
    bn_ref[...] = bn

    q = out_ref[...]
    qnorm = jnp.sqrt(jnp.sum(q * q, axis=1, keepdims=True))
    qn = q / jnp.maximum(qnorm, 1e-12)

    sim = jax.lax.dot_general(
        qn, bn, (((1,), (1,)), ((), ())), preferred_element_type=jnp.float32
    )

    base = i * bank_ref.shape[0]
    cv0 = cv_ref[...]
    ci0 = ci_ref[...]
    cv_new, ci_new, trig = _merge_fold(sim, cv0, ci0, base)
    cv_ref[...] = cv_new
    ci_ref[...] = ci_new

    @pl.when(trig)
    def _():
        cv_e, ci_e = _merge_exact(sim, cv0, ci0, base)
        cv_ref[...] = cv_e
        ci_ref[...] = ci_e

    @pl.when(i == n_steps - 1)
    def _():
        idx_ref[...] = ci_ref[...]


def _normalize_and_topk(output, bank):
    n_queries, dim = output.shape
    n_rows = bank.shape[0]
    n_tiles = n_rows // _TILE
    bank_normed, idx = pl.pallas_call(
        _scan_body,
        grid=(n_tiles,),
        in_specs=[
            pl.BlockSpec((n_queries, dim), lambda i: (0, 0)),
            pl.BlockSpec((_TILE, dim), lambda i: (i, 0)),
        ],
        out_specs=[
            pl.BlockSpec((_TILE, dim), lambda i: (i, 0)),
            pl.BlockSpec((n_queries, _CARRY_W), lambda i: (0, 0)),
        ],
        out_shape=[
            jax.ShapeDtypeStruct((n_rows, dim), jnp.float32),
            jax.ShapeDtypeStruct((n_queries, _CARRY_W), jnp.int32),
        ],
        scratch_shapes=[
            pltpu.VMEM((n_queries, _CARRY_W), jnp.float32),
            pltpu.VMEM((n_queries, _CARRY_W), jnp.int32),
        ],
        compiler_params=pltpu.CompilerParams(
            dimension_semantics=("arbitrary",),
        ),
    )(output, bank)
    return bank_normed, idx


_GATHER_ROWS = 32
_GATHER_W = 16


def _sc_gather(bank, idx_mat):
    dim = bank.shape[1]
    n_rows = idx_mat.shape[0]
    mesh = plsc.VectorSubcoreMesh(core_axis_name="core", subcore_axis_name="subcore")

    @pl.kernel(
        out_type=jax.ShapeDtypeStruct((n_rows * _GATHER_W, dim), bank.dtype),
        mesh=mesh,
    )
    def gather_kernel(x_hbm, i_hbm, o_hbm):
        def body(i_vmem, o_vmem):
            pltpu.sync_copy(x_hbm.at[i_vmem.at[0]], o_vmem)

        pltpu.emit_pipeline(
            body,
            grid=(n_rows,),
            in_specs=[pl.BlockSpec((1, _GATHER_W), index_map=lambda i: (i, 0))],
            out_specs=[pl.BlockSpec((_GATHER_W, dim), index_map=lambda i: (i, 0))],
            core_axis_name=("core", "subcore"),
            dimension_semantics=(pltpu.PARALLEL,),
        )(i_hbm, o_hbm)

    return gather_kernel(bank, idx_mat)


def kernel(output, bank):
    b, dim = output.shape
    bank_normed, idx_padded = _normalize_and_topk(output, bank)
    idx_mat = idx_padded.reshape(_GATHER_ROWS, _GATHER_W)
    gathered = _sc_gather(bank, idx_mat)
    nearest_neighbours = gathered.reshape(b, _CARRY_W, dim)[:, :_TOPK].reshape(
        -1, dim
    )
    res = jnp.concatenate(
        [output[:, None, :], nearest_neighbours.reshape(b, _TOPK, dim)], axis=1
    ).reshape(-1, dim)
    return (nearest_neighbours, res, output, bank_normed)
